# Initial kernel scaffold; baseline (speedup 1.0000x reference)
import jax
import jax.numpy as jnp
from jax.experimental import pallas as pl


def kernel(pred, target, edge_index, edge_attr, mask):
    raise NotImplementedError("write your pallas kernel here")


# speedup vs baseline: 193.0406x; 193.0406x over previous
import jax
import jax.numpy as jnp
from jax import lax
from jax.experimental import pallas as pl
from jax.experimental.pallas import tpu as pltpu
from jax.experimental.pallas import tpu_sc as plsc

N = 50000
E = 1600000
NP = 50176
STRIPE = NP // 16
CB = 5
C = CB * 128
W = 32
NCHUNK = E // C
CHUNK_BASE = NCHUNK // W
CHUNK_REM = NCHUNK % W


def _prep_body(vm_ref, va_ref, vre_ref, vim_ref):
    vm = vm_ref[...]
    va = va_ref[...]
    vre_ref[...] = vm * jnp.cos(va)
    vim_ref[...] = vm * jnp.sin(va)


def _sc_body(vre_h, vim_h, eif, eaf, zsm, o00, o01, o10, o11,
             vre, vim, srcb1, dstb1, srcb2, dstb2, attrb,
             csr, csi, cdr, cdi, reb, imb, acc_re, acc_im, sem):
    cid = lax.axis_index("c")
    sid = lax.axis_index("s")
    w = cid * 16 + sid

    r0 = sid * STRIPE
    pltpu.sync_copy(zsm, reb)
    pltpu.sync_copy(reb, acc_re.at[pl.ds(r0, STRIPE)])
    pltpu.sync_copy(reb, acc_im.at[pl.ds(r0, STRIPE)])
    pltpu.sync_copy(vre_h.at[0], vre)
    pltpu.sync_copy(vim_h.at[0], vim)
    plsc.subcore_barrier()

    nchunks = CHUNK_BASE + jnp.where(w < CHUNK_REM, 1, 0)
    chunk0 = w * CHUNK_BASE + jnp.minimum(w, CHUNK_REM)
    iota = lax.iota(jnp.int32, 16)

    def chunk_body(k, carry):
        base = (chunk0 + k) * C
        pltpu.sync_copy(eif.at[pl.ds(base, C)], srcb1)
        pltpu.sync_copy(eif.at[pl.ds(E + base, C)], dstb1)
        pltpu.sync_copy(eaf.at[pl.ds(2 * base, 2 * C)], attrb)

        def j_body(t, carry2):
            off = t * 16
            s = srcb1[pl.ds(off, 16)]
            d = dstb1[pl.ds(off, 16)]
            row2 = 2 * (off + iota)
            yre = plsc.load_gather(attrb, [row2])
            yim = plsc.load_gather(attrb, [row2 + 1])
            vsr = plsc.load_gather(vre, [s])
            vsi = plsc.load_gather(vim, [s])
            vdr = plsc.load_gather(vre, [d])
            vdi = plsc.load_gather(vim, [d])
            dr = vsr - vdr
            di = vsi - vdi
            cre = yre * dr - yim * di
            cim = yre * di + yim * dr
            i = t // 8
            loff = (t % 8) * 16
            srcb2[i, pl.ds(loff, 16)] = s
            dstb2[i, pl.ds(loff, 16)] = d
            csr[i, pl.ds(loff, 16)] = cre
            csi[i, pl.ds(loff, 16)] = cim
            cdr[i, pl.ds(loff, 16)] = -cre
            cdi[i, pl.ds(loff, 16)] = -cim
            return carry2

        lax.fori_loop(0, C // 16, j_body, 0)
        descs = []
        for i in range(CB):
            descs.append(pltpu.async_copy(
                csr.at[i], acc_re.at[srcb2.at[i]], sem, add=True))
            descs.append(pltpu.async_copy(
                csi.at[i], acc_im.at[srcb2.at[i]], sem, add=True))
            descs.append(pltpu.async_copy(
                cdr.at[i], acc_re.at[dstb2.at[i]], sem, add=True))
            descs.append(pltpu.async_copy(
                cdi.at[i], acc_im.at[dstb2.at[i]], sem, add=True))
        for desc in descs:
            desc.wait()
        return carry

    lax.fori_loop(0, nchunks, chunk_body, 0)
    plsc.subcore_barrier()

    pltpu.sync_copy(acc_re.at[pl.ds(r0, STRIPE)], reb)
    pltpu.sync_copy(acc_im.at[pl.ds(r0, STRIPE)], imb)

    @pl.when(cid == 0)
    def _():
        pltpu.sync_copy(reb, o00.at[pl.ds(r0, STRIPE)])
        pltpu.sync_copy(imb, o01.at[pl.ds(r0, STRIPE)])

    @pl.when(cid == 1)
    def _():
        pltpu.sync_copy(reb, o10.at[pl.ds(r0, STRIPE)])
        pltpu.sync_copy(imb, o11.at[pl.ds(r0, STRIPE)])


def _final_body(o00_ref, o01_ref, o10_ref, o11_ref, vre_ref, vim_ref,
                tre_ref, tim_ref, m_ref, out_ref):
    ire = o00_ref[...] + o10_ref[...]
    iim = o01_ref[...] + o11_ref[...]
    vre = vre_ref[...]
    vim = vim_ref[...]
    sre = vre * ire + vim * iim
    sim = vim * ire - vre * iim
    rre = sre - tre_ref[...]
    rim = sim - tim_ref[...]
    m = m_ref[...]
    rre = jnp.where(m, rre, 0.0)
    rim = jnp.where(m, rim, 0.0)
    a = jnp.sqrt(rre * rre + rim * rim)
    l0 = jnp.sum(a)
    l1 = jnp.sum(jnp.abs(rre))
    l2 = jnp.sum(jnp.abs(rim))
    lane = lax.broadcasted_iota(jnp.int32, (1, 128), 1)
    row = jnp.where(lane == 0, l0, jnp.where(lane == 1, l1,
                    jnp.where(lane == 2, l2, 0.0)))
    out_ref[...] = row * (1.0 / N)


_sc_call = pl.kernel(
    _sc_body,
    out_type=[jax.ShapeDtypeStruct((NP,), jnp.float32) for _ in range(4)],
    mesh=plsc.VectorSubcoreMesh(core_axis_name="c", subcore_axis_name="s",
                                num_cores=2, num_subcores=16),
    compiler_params=pltpu.CompilerParams(needs_layout_passes=False),
    scratch_types=[
        pltpu.VMEM((NP,), jnp.float32),
        pltpu.VMEM((NP,), jnp.float32),
        pltpu.VMEM((C,), jnp.int32),
        pltpu.VMEM((C,), jnp.int32),
        pltpu.VMEM((CB, 128), jnp.int32),
        pltpu.VMEM((CB, 128), jnp.int32),
        pltpu.VMEM((2 * C,), jnp.float32),
        pltpu.VMEM((CB, 128), jnp.float32),
        pltpu.VMEM((CB, 128), jnp.float32),
        pltpu.VMEM((CB, 128), jnp.float32),
        pltpu.VMEM((CB, 128), jnp.float32),
        pltpu.VMEM((STRIPE,), jnp.float32),
        pltpu.VMEM((STRIPE,), jnp.float32),
        pltpu.VMEM_SHARED((NP,), jnp.float32),
        pltpu.VMEM_SHARED((NP,), jnp.float32),
        pltpu.SemaphoreType.DMA,
    ],
)


def kernel(pred, target, edge_index, edge_attr, mask):
    pad = (0, NP - N)
    vm = jnp.pad(pred[:, 0], pad).reshape(1, NP)
    va = jnp.pad(pred[:, 1], pad).reshape(1, NP)
    tre = jnp.pad(target[:, 0], pad).reshape(1, NP)
    tim = jnp.pad(target[:, 1], pad).reshape(1, NP)
    mp = jnp.pad(mask, pad).reshape(1, NP)
    eif = edge_index.reshape(2 * E)
    eaf = edge_attr.reshape(2 * E)
    zsm = jnp.zeros((STRIPE,), jnp.float32)

    vre_h, vim_h = pl.pallas_call(
        _prep_body,
        out_shape=[jax.ShapeDtypeStruct((1, NP), jnp.float32)] * 2,
    )(vm, va)

    o00, o01, o10, o11 = _sc_call(vre_h, vim_h, eif, eaf, zsm)

    out = pl.pallas_call(
        _final_body,
        out_shape=jax.ShapeDtypeStruct((1, 128), jnp.float32),
    )(o00.reshape(1, NP), o01.reshape(1, NP), o10.reshape(1, NP),
      o11.reshape(1, NP), vre_h, vim_h, tre, tim, mp)
    return out[0, :3]


# speedup vs baseline: 989.8787x; 5.1278x over previous
import jax
import jax.numpy as jnp
from jax import lax
from jax.experimental import pallas as pl
from jax.experimental.pallas import tpu as pltpu
from jax.experimental.pallas import tpu_sc as plsc

N = 50000
E = 1600000
NP = 50176
STRIPE = NP // 16
CB = 5
C = CB * 128
W = 32
NCHUNK = E // C
CHUNK_BASE = NCHUNK // W
CHUNK_REM = NCHUNK % W


def _prep_body(vm_ref, va_ref, vre_ref, vim_ref):
    vm = vm_ref[...]
    va = va_ref[...]
    vre_ref[...] = vm * jnp.cos(va)
    vim_ref[...] = vm * jnp.sin(va)


def _sc_body(vre_h, vim_h, src_h, dst_h, yre_h, yim_h, zsm,
             o00, o01, o10, o11,
             vre, vim, srcb1, dstb1, srcb2, dstb2, yreb, yimb,
             csr, csi, cdr, cdi, reb, imb, acc_re, acc_im, sem):
    cid = lax.axis_index("c")
    sid = lax.axis_index("s")
    w = cid * 16 + sid

    r0 = sid * STRIPE
    pltpu.sync_copy(zsm, reb)
    pltpu.sync_copy(reb, acc_re.at[pl.ds(r0, STRIPE)])
    pltpu.sync_copy(reb, acc_im.at[pl.ds(r0, STRIPE)])
    pltpu.sync_copy(vre_h.at[0], vre)
    pltpu.sync_copy(vim_h.at[0], vim)
    plsc.subcore_barrier()

    nchunks = CHUNK_BASE + jnp.where(w < CHUNK_REM, 1, 0)
    chunk0 = w * CHUNK_BASE + jnp.minimum(w, CHUNK_REM)
    iota = lax.iota(jnp.int32, 16)

    def chunk_body(k, carry):
        base = (chunk0 + k) * C
        pltpu.sync_copy(src_h.at[pl.ds(base, C)], srcb1)
        pltpu.sync_copy(dst_h.at[pl.ds(base, C)], dstb1)
        pltpu.sync_copy(yre_h.at[pl.ds(base, C)], yreb)
        pltpu.sync_copy(yim_h.at[pl.ds(base, C)], yimb)

        def j_body(t, carry2):
            off = t * 16
            s = srcb1[pl.ds(off, 16)]
            d = dstb1[pl.ds(off, 16)]
            yre = yreb[pl.ds(off, 16)]
            yim = yimb[pl.ds(off, 16)]
            vsr = plsc.load_gather(vre, [s])
            vsi = plsc.load_gather(vim, [s])
            vdr = plsc.load_gather(vre, [d])
            vdi = plsc.load_gather(vim, [d])
            dr = vsr - vdr
            di = vsi - vdi
            cre = yre * dr - yim * di
            cim = yre * di + yim * dr
            i = t // 8
            loff = (t % 8) * 16
            srcb2[i, pl.ds(loff, 16)] = s
            dstb2[i, pl.ds(loff, 16)] = d
            csr[i, pl.ds(loff, 16)] = cre
            csi[i, pl.ds(loff, 16)] = cim
            cdr[i, pl.ds(loff, 16)] = -cre
            cdi[i, pl.ds(loff, 16)] = -cim
            return carry2

        lax.fori_loop(0, C // 16, j_body, 0)
        descs = []
        for i in range(CB):
            descs.append(pltpu.async_copy(
                csr.at[i], acc_re.at[srcb2.at[i]], sem, add=True))
            descs.append(pltpu.async_copy(
                csi.at[i], acc_im.at[srcb2.at[i]], sem, add=True))
            descs.append(pltpu.async_copy(
                cdr.at[i], acc_re.at[dstb2.at[i]], sem, add=True))
            descs.append(pltpu.async_copy(
                cdi.at[i], acc_im.at[dstb2.at[i]], sem, add=True))
        for desc in descs:
            desc.wait()
        return carry

    lax.fori_loop(0, nchunks, chunk_body, 0)
    plsc.subcore_barrier()

    pltpu.sync_copy(acc_re.at[pl.ds(r0, STRIPE)], reb)
    pltpu.sync_copy(acc_im.at[pl.ds(r0, STRIPE)], imb)

    @pl.when(cid == 0)
    def _():
        pltpu.sync_copy(reb, o00.at[pl.ds(r0, STRIPE)])
        pltpu.sync_copy(imb, o01.at[pl.ds(r0, STRIPE)])

    @pl.when(cid == 1)
    def _():
        pltpu.sync_copy(reb, o10.at[pl.ds(r0, STRIPE)])
        pltpu.sync_copy(imb, o11.at[pl.ds(r0, STRIPE)])


def _final_body(o00_ref, o01_ref, o10_ref, o11_ref, vre_ref, vim_ref,
                tre_ref, tim_ref, m_ref, out_ref):
    ire = o00_ref[...] + o10_ref[...]
    iim = o01_ref[...] + o11_ref[...]
    vre = vre_ref[...]
    vim = vim_ref[...]
    sre = vre * ire + vim * iim
    sim = vim * ire - vre * iim
    rre = sre - tre_ref[...]
    rim = sim - tim_ref[...]
    m = m_ref[...]
    rre = jnp.where(m, rre, 0.0)
    rim = jnp.where(m, rim, 0.0)
    a = jnp.sqrt(rre * rre + rim * rim)
    l0 = jnp.sum(a)
    l1 = jnp.sum(jnp.abs(rre))
    l2 = jnp.sum(jnp.abs(rim))
    lane = lax.broadcasted_iota(jnp.int32, (1, 128), 1)
    row = jnp.where(lane == 0, l0, jnp.where(lane == 1, l1,
                    jnp.where(lane == 2, l2, 0.0)))
    out_ref[...] = row * (1.0 / N)


_sc_call = pl.kernel(
    _sc_body,
    out_type=[jax.ShapeDtypeStruct((NP,), jnp.float32) for _ in range(4)],
    mesh=plsc.VectorSubcoreMesh(core_axis_name="c", subcore_axis_name="s",
                                num_cores=2, num_subcores=16),
    compiler_params=pltpu.CompilerParams(needs_layout_passes=False),
    scratch_types=[
        pltpu.VMEM((NP,), jnp.float32),
        pltpu.VMEM((NP,), jnp.float32),
        pltpu.VMEM((C,), jnp.int32),
        pltpu.VMEM((C,), jnp.int32),
        pltpu.VMEM((CB, 128), jnp.int32),
        pltpu.VMEM((CB, 128), jnp.int32),
        pltpu.VMEM((C,), jnp.float32),
        pltpu.VMEM((C,), jnp.float32),
        pltpu.VMEM((CB, 128), jnp.float32),
        pltpu.VMEM((CB, 128), jnp.float32),
        pltpu.VMEM((CB, 128), jnp.float32),
        pltpu.VMEM((CB, 128), jnp.float32),
        pltpu.VMEM((STRIPE,), jnp.float32),
        pltpu.VMEM((STRIPE,), jnp.float32),
        pltpu.VMEM_SHARED((NP,), jnp.float32),
        pltpu.VMEM_SHARED((NP,), jnp.float32),
        pltpu.SemaphoreType.DMA,
    ],
)


def kernel(pred, target, edge_index, edge_attr, mask):
    pad = (0, NP - N)
    vm = jnp.pad(pred[:, 0], pad).reshape(1, NP)
    va = jnp.pad(pred[:, 1], pad).reshape(1, NP)
    tre = jnp.pad(target[:, 0], pad).reshape(1, NP)
    tim = jnp.pad(target[:, 1], pad).reshape(1, NP)
    mp = jnp.pad(mask, pad).reshape(1, NP)
    src = edge_index[0]
    dst = edge_index[1]
    yre_a = edge_attr[:, 0]
    yim_a = edge_attr[:, 1]
    zsm = jnp.zeros((STRIPE,), jnp.float32)

    vre_h, vim_h = pl.pallas_call(
        _prep_body,
        out_shape=[jax.ShapeDtypeStruct((1, NP), jnp.float32)] * 2,
    )(vm, va)

    o00, o01, o10, o11 = _sc_call(vre_h, vim_h, src, dst, yre_a, yim_a, zsm)

    out = pl.pallas_call(
        _final_body,
        out_shape=jax.ShapeDtypeStruct((1, 128), jnp.float32),
    )(o00.reshape(1, NP), o01.reshape(1, NP), o10.reshape(1, NP),
      o11.reshape(1, NP), vre_h, vim_h, tre, tim, mp)
    return out[0, :3]


# speedup vs baseline: 1838.2527x; 1.8570x over previous
import jax
import jax.numpy as jnp
from jax import lax
from jax.experimental import pallas as pl
from jax.experimental.pallas import tpu as pltpu
from jax.experimental.pallas import tpu_sc as plsc

N = 50000
E = 1600000
NP = 50176
STRIPE = NP // 16
CB = 5
C = CB * 128
W = 32
NCHUNK = E // C
CHUNK_BASE = NCHUNK // W
CHUNK_REM = NCHUNK % W


def _prep_body(vm_ref, va_ref, vre_ref, vim_ref):
    vm = vm_ref[...]
    va = va_ref[...]
    vre_ref[...] = vm * jnp.cos(va)
    vim_ref[...] = vm * jnp.sin(va)


def _sc_body(vre_h, vim_h, src_h, dst_h, yre_h, yim_h, zsm,
             o00, o01, o10, o11,
             vre, vim,
             srcb1_0, dstb1_0, yreb_0, yimb_0,
             srcb1_1, dstb1_1, yreb_1, yimb_1,
             srcb2_0, dstb2_0, csr_0, csi_0, cdr_0, cdi_0,
             srcb2_1, dstb2_1, csr_1, csi_1, cdr_1, cdi_1,
             reb, imb, acc_re, acc_im,
             vsem, isem0, isem1, ssem0, ssem1):
    cid = lax.axis_index("c")
    sid = lax.axis_index("s")
    w = cid * 16 + sid
    r0 = sid * STRIPE
    iota = lax.iota(jnp.int32, 16)
    chunk0 = w * CHUNK_BASE

    INB = [(srcb1_0, dstb1_0, yreb_0, yimb_0),
           (srcb1_1, dstb1_1, yreb_1, yimb_1)]
    STG = [(srcb2_0, dstb2_0, csr_0, csi_0, cdr_0, cdi_0),
           (srcb2_1, dstb2_1, csr_1, csi_1, cdr_1, cdi_1)]
    ISEM = [isem0, isem1]
    SSEM = [ssem0, ssem1]

    def in_descs(g, p):
        base = g * C
        sb, db, yb, ib = INB[p]
        return [
            pltpu.make_async_copy(src_h.at[pl.ds(base, C)], sb, ISEM[p]),
            pltpu.make_async_copy(dst_h.at[pl.ds(base, C)], db, ISEM[p]),
            pltpu.make_async_copy(yre_h.at[pl.ds(base, C)], yb, ISEM[p]),
            pltpu.make_async_copy(yim_h.at[pl.ds(base, C)], ib, ISEM[p]),
        ]

    def sc_descs(p):
        s2, d2, cr, ci, dr_, di_ = STG[p]
        out = []
        for i in range(CB):
            out.append(pltpu.make_async_copy(
                cr.at[i], acc_re.at[s2.at[i]], SSEM[p]))
            out.append(pltpu.make_async_copy(
                ci.at[i], acc_im.at[s2.at[i]], SSEM[p]))
            out.append(pltpu.make_async_copy(
                dr_.at[i], acc_re.at[d2.at[i]], SSEM[p]))
            out.append(pltpu.make_async_copy(
                di_.at[i], acc_im.at[d2.at[i]], SSEM[p]))
        return out

    def compute(p):
        sb, db, yb, ib = INB[p]
        s2, d2, cr, ci, dr_, di_ = STG[p]

        def j_body(t, carry2):
            off = t * 16
            s = sb[pl.ds(off, 16)]
            d = db[pl.ds(off, 16)]
            yre = yb[pl.ds(off, 16)]
            yim = ib[pl.ds(off, 16)]
            vsr = plsc.load_gather(vre, [s])
            vsi = plsc.load_gather(vim, [s])
            vdr = plsc.load_gather(vre, [d])
            vdi = plsc.load_gather(vim, [d])
            dre = vsr - vdr
            dim = vsi - vdi
            cre = yre * dre - yim * dim
            cim = yre * dim + yim * dre
            i = t // 8
            loff = (t % 8) * 16
            s2[i, pl.ds(loff, 16)] = s
            d2[i, pl.ds(loff, 16)] = d
            cr[i, pl.ds(loff, 16)] = cre
            ci[i, pl.ds(loff, 16)] = cim
            dr_[i, pl.ds(loff, 16)] = -cre
            di_[i, pl.ds(loff, 16)] = -cim
            return carry2

        lax.fori_loop(0, C // 16, j_body, 0)

    vdesc = [pltpu.make_async_copy(vre_h.at[0], vre, vsem),
             pltpu.make_async_copy(vim_h.at[0], vim, vsem)]
    for d in vdesc:
        d.start()
    for d in in_descs(chunk0, 0):
        d.start()
    for d in in_descs(chunk0 + 1, 1):
        d.start()
    pltpu.sync_copy(zsm, reb)
    pltpu.sync_copy(reb, acc_re.at[pl.ds(r0, STRIPE)])
    pltpu.sync_copy(reb, acc_im.at[pl.ds(r0, STRIPE)])
    for d in vdesc:
        d.wait()
    plsc.subcore_barrier()

    def phase(L, p):
        for d in in_descs(chunk0 + L, p):
            d.wait()

        @pl.when(L >= 2)
        def _():
            for d in sc_descs(p):
                d.wait()

        compute(p)
        for d in sc_descs(p):
            d.start(add=True)

        @pl.when(L + 2 < CHUNK_BASE)
        def _():
            for d in in_descs(chunk0 + L + 2, p):
                d.start()

    def body2(k2, carry):
        phase(2 * k2, 0)
        phase(2 * k2 + 1, 1)
        return carry

    lax.fori_loop(0, CHUNK_BASE // 2, body2, 0)
    for d in sc_descs(0):
        d.wait()
    for d in sc_descs(1):
        d.wait()

    @pl.when(w < CHUNK_REM)
    def _():
        g = W * CHUNK_BASE + w
        for d in in_descs(g, 0):
            d.start()
        for d in in_descs(g, 0):
            d.wait()
        compute(0)
        for d in sc_descs(0):
            d.start(add=True)
        for d in sc_descs(0):
            d.wait()

    plsc.subcore_barrier()

    pltpu.sync_copy(acc_re.at[pl.ds(r0, STRIPE)], reb)
    pltpu.sync_copy(acc_im.at[pl.ds(r0, STRIPE)], imb)

    @pl.when(cid == 0)
    def _():
        pltpu.sync_copy(reb, o00.at[pl.ds(r0, STRIPE)])
        pltpu.sync_copy(imb, o01.at[pl.ds(r0, STRIPE)])

    @pl.when(cid == 1)
    def _():
        pltpu.sync_copy(reb, o10.at[pl.ds(r0, STRIPE)])
        pltpu.sync_copy(imb, o11.at[pl.ds(r0, STRIPE)])


def _final_body(o00_ref, o01_ref, o10_ref, o11_ref, vre_ref, vim_ref,
                tre_ref, tim_ref, m_ref, out_ref):
    ire = o00_ref[...] + o10_ref[...]
    iim = o01_ref[...] + o11_ref[...]
    vre = vre_ref[...]
    vim = vim_ref[...]
    sre = vre * ire + vim * iim
    sim = vim * ire - vre * iim
    rre = sre - tre_ref[...]
    rim = sim - tim_ref[...]
    m = m_ref[...]
    rre = jnp.where(m, rre, 0.0)
    rim = jnp.where(m, rim, 0.0)
    a = jnp.sqrt(rre * rre + rim * rim)
    l0 = jnp.sum(a)
    l1 = jnp.sum(jnp.abs(rre))
    l2 = jnp.sum(jnp.abs(rim))
    lane = lax.broadcasted_iota(jnp.int32, (1, 128), 1)
    row = jnp.where(lane == 0, l0, jnp.where(lane == 1, l1,
                    jnp.where(lane == 2, l2, 0.0)))
    out_ref[...] = row * (1.0 / N)


_sc_call = pl.kernel(
    _sc_body,
    out_type=[jax.ShapeDtypeStruct((NP,), jnp.float32) for _ in range(4)],
    mesh=plsc.VectorSubcoreMesh(core_axis_name="c", subcore_axis_name="s",
                                num_cores=2, num_subcores=16),
    compiler_params=pltpu.CompilerParams(needs_layout_passes=False),
    scratch_types=[
        pltpu.VMEM((NP,), jnp.float32),
        pltpu.VMEM((NP,), jnp.float32),
        pltpu.VMEM((C,), jnp.int32),
        pltpu.VMEM((C,), jnp.int32),
        pltpu.VMEM((C,), jnp.float32),
        pltpu.VMEM((C,), jnp.float32),
        pltpu.VMEM((C,), jnp.int32),
        pltpu.VMEM((C,), jnp.int32),
        pltpu.VMEM((C,), jnp.float32),
        pltpu.VMEM((C,), jnp.float32),
        pltpu.VMEM((CB, 128), jnp.int32),
        pltpu.VMEM((CB, 128), jnp.int32),
        pltpu.VMEM((CB, 128), jnp.float32),
        pltpu.VMEM((CB, 128), jnp.float32),
        pltpu.VMEM((CB, 128), jnp.float32),
        pltpu.VMEM((CB, 128), jnp.float32),
        pltpu.VMEM((CB, 128), jnp.int32),
        pltpu.VMEM((CB, 128), jnp.int32),
        pltpu.VMEM((CB, 128), jnp.float32),
        pltpu.VMEM((CB, 128), jnp.float32),
        pltpu.VMEM((CB, 128), jnp.float32),
        pltpu.VMEM((CB, 128), jnp.float32),
        pltpu.VMEM((STRIPE,), jnp.float32),
        pltpu.VMEM((STRIPE,), jnp.float32),
        pltpu.VMEM_SHARED((NP,), jnp.float32),
        pltpu.VMEM_SHARED((NP,), jnp.float32),
        pltpu.SemaphoreType.DMA,
        pltpu.SemaphoreType.DMA,
        pltpu.SemaphoreType.DMA,
        pltpu.SemaphoreType.DMA,
        pltpu.SemaphoreType.DMA,
    ],
)


def kernel(pred, target, edge_index, edge_attr, mask):
    pad = (0, NP - N)
    vm = jnp.pad(pred[:, 0], pad).reshape(1, NP)
    va = jnp.pad(pred[:, 1], pad).reshape(1, NP)
    tre = jnp.pad(target[:, 0], pad).reshape(1, NP)
    tim = jnp.pad(target[:, 1], pad).reshape(1, NP)
    mp = jnp.pad(mask, pad).reshape(1, NP)
    src = edge_index[0]
    dst = edge_index[1]
    yre_a = edge_attr[:, 0]
    yim_a = edge_attr[:, 1]
    zsm = jnp.zeros((STRIPE,), jnp.float32)

    vre_h, vim_h = pl.pallas_call(
        _prep_body,
        out_shape=[jax.ShapeDtypeStruct((1, NP), jnp.float32)] * 2,
    )(vm, va)

    o00, o01, o10, o11 = _sc_call(vre_h, vim_h, src, dst, yre_a, yim_a, zsm)

    out = pl.pallas_call(
        _final_body,
        out_shape=jax.ShapeDtypeStruct((1, 128), jnp.float32),
    )(o00.reshape(1, NP), o01.reshape(1, NP), o10.reshape(1, NP),
      o11.reshape(1, NP), vre_h, vim_h, tre, tim, mp)
    return out[0, :3]


# speedup vs baseline: 2569.3423x; 1.3977x over previous
import jax
import jax.numpy as jnp
from jax import lax
from jax.experimental import pallas as pl
from jax.experimental.pallas import tpu as pltpu
from jax.experimental.pallas import tpu_sc as plsc

N = 50000
E = 1600000
NP = 50176
STRIPE = NP // 16
CB = 4
C = CB * 128
W = 32
NCHUNK = E // C
CHUNK_BASE = NCHUNK // W
CHUNK_REM = NCHUNK % W


def _prep_body(vm_ref, va_ref, vre_ref, vim_ref):
    vm = vm_ref[...]
    va = va_ref[...]
    vre_ref[...] = vm * jnp.cos(va)
    vim_ref[...] = vm * jnp.sin(va)


def _sc_body(ei2_h, ea2_h, vre_h, vim_h, zsm,
             o00, o01, o10, o11,
             vre, vim,
             eib_0, attrb_0,
             eib_1, attrb_1,
             srcb2_0, dstb2_0, csr_0, csi_0, cdr_0, cdi_0,
             srcb2_1, dstb2_1, csr_1, csi_1, cdr_1, cdi_1,
             reb, imb, acc_re, acc_im,
             vsem, isem0, isem1, ssem0, ssem1):
    cid = lax.axis_index("c")
    sid = lax.axis_index("s")
    w = cid * 16 + sid
    r0 = sid * STRIPE
    iota = lax.iota(jnp.int32, 16)
    chunk0 = w * CHUNK_BASE

    INB = [(eib_0, attrb_0), (eib_1, attrb_1)]
    STG = [(srcb2_0, dstb2_0, csr_0, csi_0, cdr_0, cdi_0),
           (srcb2_1, dstb2_1, csr_1, csi_1, cdr_1, cdi_1)]
    ISEM = [isem0, isem1]
    SSEM = [ssem0, ssem1]

    def in_descs(g, p):
        eb, ab = INB[p]
        return [
            pltpu.make_async_copy(ei2_h.at[pl.ds(g * 8, 8)], eb, ISEM[p]),
            pltpu.make_async_copy(ea2_h.at[pl.ds(g * 8, 8)], ab, ISEM[p]),
        ]

    def sc_descs(p):
        s2, d2, cr, ci, dr_, di_ = STG[p]
        out = []
        for i in range(CB):
            out.append(pltpu.make_async_copy(
                cr.at[i], acc_re.at[s2.at[i]], SSEM[p]))
            out.append(pltpu.make_async_copy(
                ci.at[i], acc_im.at[s2.at[i]], SSEM[p]))
            out.append(pltpu.make_async_copy(
                dr_.at[i], acc_re.at[d2.at[i]], SSEM[p]))
            out.append(pltpu.make_async_copy(
                di_.at[i], acc_im.at[d2.at[i]], SSEM[p]))
        return out

    def compute(p):
        eb, ab = INB[p]
        s2, d2, cr, ci, dr_, di_ = STG[p]

        def j_body(t, carry2):
            blk = 2 * (t // 8)
            loff = (t % 8) * 16
            s = eb[blk, pl.ds(loff, 16)]
            d = eb[blk + 1, pl.ds(loff, 16)]
            yre = ab[blk, pl.ds(loff, 16)]
            yim = ab[blk + 1, pl.ds(loff, 16)]
            vsr = plsc.load_gather(vre, [s])
            vsi = plsc.load_gather(vim, [s])
            vdr = plsc.load_gather(vre, [d])
            vdi = plsc.load_gather(vim, [d])
            dre = vsr - vdr
            dim = vsi - vdi
            cre = yre * dre - yim * dim
            cim = yre * dim + yim * dre
            i = t // 8
            s2[i, pl.ds(loff, 16)] = s
            d2[i, pl.ds(loff, 16)] = d
            cr[i, pl.ds(loff, 16)] = cre
            ci[i, pl.ds(loff, 16)] = cim
            dr_[i, pl.ds(loff, 16)] = -cre
            di_[i, pl.ds(loff, 16)] = -cim
            return carry2

        lax.fori_loop(0, C // 16, j_body, 0)

    vdesc = [pltpu.make_async_copy(vre_h.at[0], vre, vsem),
             pltpu.make_async_copy(vim_h.at[0], vim, vsem)]
    for d in vdesc:
        d.start()
    for d in in_descs(chunk0, 0):
        d.start()
    for d in in_descs(chunk0 + 1, 1):
        d.start()
    pltpu.sync_copy(zsm, reb)
    pltpu.sync_copy(reb, acc_re.at[pl.ds(r0, STRIPE)])
    pltpu.sync_copy(reb, acc_im.at[pl.ds(r0, STRIPE)])
    for d in vdesc:
        d.wait()
    plsc.subcore_barrier()

    def phase(L, p):
        for d in in_descs(chunk0 + L, p):
            d.wait()

        @pl.when(L >= 2)
        def _():
            for d in sc_descs(p):
                d.wait()

        compute(p)
        for d in sc_descs(p):
            d.start(add=True)

        @pl.when(L + 2 < CHUNK_BASE)
        def _():
            for d in in_descs(chunk0 + L + 2, p):
                d.start()

    def body2(k2, carry):
        phase(2 * k2, 0)
        phase(2 * k2 + 1, 1)
        return carry

    lax.fori_loop(0, CHUNK_BASE // 2, body2, 0)
    if CHUNK_BASE % 2:
        phase(jnp.int32(CHUNK_BASE - 1), 0)
    for d in sc_descs(0):
        d.wait()
    for d in sc_descs(1):
        d.wait()

    @pl.when(w < CHUNK_REM)
    def _():
        g = W * CHUNK_BASE + w
        for d in in_descs(g, 0):
            d.start()
        for d in in_descs(g, 0):
            d.wait()
        compute(0)
        for d in sc_descs(0):
            d.start(add=True)
        for d in sc_descs(0):
            d.wait()

    plsc.subcore_barrier()

    pltpu.sync_copy(acc_re.at[pl.ds(r0, STRIPE)], reb)
    pltpu.sync_copy(acc_im.at[pl.ds(r0, STRIPE)], imb)

    @pl.when(cid == 0)
    def _():
        pltpu.sync_copy(reb, o00.at[pl.ds(r0, STRIPE)])
        pltpu.sync_copy(imb, o01.at[pl.ds(r0, STRIPE)])

    @pl.when(cid == 1)
    def _():
        pltpu.sync_copy(reb, o10.at[pl.ds(r0, STRIPE)])
        pltpu.sync_copy(imb, o11.at[pl.ds(r0, STRIPE)])


def _final_body(o00_ref, o01_ref, o10_ref, o11_ref, vre_ref, vim_ref,
                tre_ref, tim_ref, m_ref, out_ref):
    ire = o00_ref[...] + o10_ref[...]
    iim = o01_ref[...] + o11_ref[...]
    vre = vre_ref[...]
    vim = vim_ref[...]
    sre = vre * ire + vim * iim
    sim = vim * ire - vre * iim
    rre = sre - tre_ref[...]
    rim = sim - tim_ref[...]
    m = m_ref[...]
    rre = jnp.where(m, rre, 0.0)
    rim = jnp.where(m, rim, 0.0)
    a = jnp.sqrt(rre * rre + rim * rim)
    l0 = jnp.sum(a)
    l1 = jnp.sum(jnp.abs(rre))
    l2 = jnp.sum(jnp.abs(rim))
    lane = lax.broadcasted_iota(jnp.int32, (1, 128), 1)
    row = jnp.where(lane == 0, l0, jnp.where(lane == 1, l1,
                    jnp.where(lane == 2, l2, 0.0)))
    out_ref[...] = row * (1.0 / N)


_sc_call = pl.kernel(
    _sc_body,
    out_type=[jax.ShapeDtypeStruct((NP,), jnp.float32) for _ in range(4)],
    mesh=plsc.VectorSubcoreMesh(core_axis_name="c", subcore_axis_name="s",
                                num_cores=2, num_subcores=16),
    compiler_params=pltpu.CompilerParams(needs_layout_passes=False),
    scratch_types=[
        pltpu.VMEM((NP,), jnp.float32),
        pltpu.VMEM((NP,), jnp.float32),
        pltpu.VMEM((8, 128), jnp.int32),
        pltpu.VMEM((8, 128), jnp.float32),
        pltpu.VMEM((8, 128), jnp.int32),
        pltpu.VMEM((8, 128), jnp.float32),
        pltpu.VMEM((CB, 128), jnp.int32),
        pltpu.VMEM((CB, 128), jnp.int32),
        pltpu.VMEM((CB, 128), jnp.float32),
        pltpu.VMEM((CB, 128), jnp.float32),
        pltpu.VMEM((CB, 128), jnp.float32),
        pltpu.VMEM((CB, 128), jnp.float32),
        pltpu.VMEM((CB, 128), jnp.int32),
        pltpu.VMEM((CB, 128), jnp.int32),
        pltpu.VMEM((CB, 128), jnp.float32),
        pltpu.VMEM((CB, 128), jnp.float32),
        pltpu.VMEM((CB, 128), jnp.float32),
        pltpu.VMEM((CB, 128), jnp.float32),
        pltpu.VMEM((STRIPE,), jnp.float32),
        pltpu.VMEM((STRIPE,), jnp.float32),
        pltpu.VMEM_SHARED((NP,), jnp.float32),
        pltpu.VMEM_SHARED((NP,), jnp.float32),
        pltpu.SemaphoreType.DMA,
        pltpu.SemaphoreType.DMA,
        pltpu.SemaphoreType.DMA,
        pltpu.SemaphoreType.DMA,
        pltpu.SemaphoreType.DMA,
    ],
)


def kernel(pred, target, edge_index, edge_attr, mask):
    pad = (0, NP - N)
    vm = jnp.pad(pred[:, 0], pad).reshape(1, NP)
    va = jnp.pad(pred[:, 1], pad).reshape(1, NP)
    tre = jnp.pad(target[:, 0], pad).reshape(1, NP)
    tim = jnp.pad(target[:, 1], pad).reshape(1, NP)
    mp = jnp.pad(mask, pad).reshape(1, NP)
    ei2 = (edge_index.reshape(2, E // 128, 128)
           .transpose(1, 0, 2).reshape(E // 64, 128))
    ea2 = (edge_attr.reshape(E // 128, 128, 2)
           .transpose(0, 2, 1).reshape(E // 64, 128))
    zsm = jnp.zeros((STRIPE,), jnp.float32)

    vre_h, vim_h = pl.pallas_call(
        _prep_body,
        out_shape=[jax.ShapeDtypeStruct((1, NP), jnp.float32)] * 2,
    )(vm, va)

    o00, o01, o10, o11 = _sc_call(ei2, ea2, vre_h, vim_h, zsm)

    out = pl.pallas_call(
        _final_body,
        out_shape=jax.ShapeDtypeStruct((1, 128), jnp.float32),
    )(o00.reshape(1, NP), o01.reshape(1, NP), o10.reshape(1, NP),
      o11.reshape(1, NP), vre_h, vim_h, tre, tim, mp)
    return out[0, :3]


# speedup vs baseline: 2707.3151x; 1.0537x over previous
import jax
import jax.numpy as jnp
from jax import lax
from jax.experimental import pallas as pl
from jax.experimental.pallas import tpu as pltpu
from jax.experimental.pallas import tpu_sc as plsc

N = 50000
E = 1600000
NP = 50176
STRIPE = NP // 16
CB = 4
C = CB * 128
W = 32
NCHUNK = E // C
CHUNK_BASE = NCHUNK // W
CHUNK_REM = NCHUNK % W


def _prep_body(vm_ref, va_ref, vre_ref, vim_ref):
    vm = vm_ref[...]
    va = va_ref[...]
    vre_ref[...] = vm * jnp.cos(va)
    vim_ref[...] = vm * jnp.sin(va)


def _sc_body(ei2_h, ea2_h, vre_h, vim_h, zsm,
             o00, o01, o10, o11,
             vre, vim,
             eib_0, attrb_0,
             eib_1, attrb_1,
             srcb2_0, dstb2_0, csr_0, csi_0, cdr_0, cdi_0,
             srcb2_1, dstb2_1, csr_1, csi_1, cdr_1, cdi_1,
             reb, imb, acc_re, acc_im,
             vsem, isem0, isem1, ssem0, ssem1):
    cid = lax.axis_index("c")
    sid = lax.axis_index("s")
    w = cid * 16 + sid
    r0 = sid * STRIPE
    iota = lax.iota(jnp.int32, 16)
    chunk0 = w * CHUNK_BASE

    INB = [(eib_0, attrb_0), (eib_1, attrb_1)]
    STG = [(srcb2_0, dstb2_0, csr_0, csi_0, cdr_0, cdi_0),
           (srcb2_1, dstb2_1, csr_1, csi_1, cdr_1, cdi_1)]
    ISEM = [isem0, isem1]
    SSEM = [ssem0, ssem1]

    def in_descs(g, p):
        eb, ab = INB[p]
        return [
            pltpu.make_async_copy(ei2_h.at[pl.ds(g * 8, 8)], eb, ISEM[p]),
            pltpu.make_async_copy(ea2_h.at[pl.ds(g * 8, 8)], ab, ISEM[p]),
        ]

    def sc_descs(p):
        s2, d2, cr, ci, dr_, di_ = STG[p]
        out = []
        for i in range(CB):
            out.append(pltpu.make_async_copy(
                cr.at[i], acc_re.at[s2.at[i]], SSEM[p]))
            out.append(pltpu.make_async_copy(
                ci.at[i], acc_im.at[s2.at[i]], SSEM[p]))
            out.append(pltpu.make_async_copy(
                dr_.at[i], acc_re.at[d2.at[i]], SSEM[p]))
            out.append(pltpu.make_async_copy(
                di_.at[i], acc_im.at[d2.at[i]], SSEM[p]))
        return out

    def compute(p):
        eb, ab = INB[p]
        s2, d2, cr, ci, dr_, di_ = STG[p]

        @plsc.parallel_loop(0, C // 16)
        def j_body(t):
            blk = 2 * (t // 8)
            loff = (t % 8) * 16
            s = eb[blk, pl.ds(loff, 16)]
            d = eb[blk + 1, pl.ds(loff, 16)]
            yre = ab[blk, pl.ds(loff, 16)]
            yim = ab[blk + 1, pl.ds(loff, 16)]
            vsr = plsc.load_gather(vre, [s])
            vsi = plsc.load_gather(vim, [s])
            vdr = plsc.load_gather(vre, [d])
            vdi = plsc.load_gather(vim, [d])
            dre = vsr - vdr
            dim = vsi - vdi
            cre = yre * dre - yim * dim
            cim = yre * dim + yim * dre
            i = t // 8
            s2[i, pl.ds(loff, 16)] = s
            d2[i, pl.ds(loff, 16)] = d
            cr[i, pl.ds(loff, 16)] = cre
            ci[i, pl.ds(loff, 16)] = cim
            dr_[i, pl.ds(loff, 16)] = -cre
            di_[i, pl.ds(loff, 16)] = -cim

    vdesc = [pltpu.make_async_copy(vre_h.at[0], vre, vsem),
             pltpu.make_async_copy(vim_h.at[0], vim, vsem)]
    for d in vdesc:
        d.start()
    for d in in_descs(chunk0, 0):
        d.start()
    for d in in_descs(chunk0 + 1, 1):
        d.start()
    pltpu.sync_copy(zsm, reb)
    pltpu.sync_copy(reb, acc_re.at[pl.ds(r0, STRIPE)])
    pltpu.sync_copy(reb, acc_im.at[pl.ds(r0, STRIPE)])
    for d in vdesc:
        d.wait()
    plsc.subcore_barrier()

    def phase(L, p):
        for d in in_descs(chunk0 + L, p):
            d.wait()

        @pl.when(L >= 2)
        def _():
            for d in sc_descs(p):
                d.wait()

        compute(p)
        for d in sc_descs(p):
            d.start(add=True)

        @pl.when(L + 2 < CHUNK_BASE)
        def _():
            for d in in_descs(chunk0 + L + 2, p):
                d.start()

    def body2(k2, carry):
        phase(2 * k2, 0)
        phase(2 * k2 + 1, 1)
        return carry

    lax.fori_loop(0, CHUNK_BASE // 2, body2, 0)
    if CHUNK_BASE % 2:
        phase(jnp.int32(CHUNK_BASE - 1), 0)
    for d in sc_descs(0):
        d.wait()
    for d in sc_descs(1):
        d.wait()

    @pl.when(w < CHUNK_REM)
    def _():
        g = W * CHUNK_BASE + w
        for d in in_descs(g, 0):
            d.start()
        for d in in_descs(g, 0):
            d.wait()
        compute(0)
        for d in sc_descs(0):
            d.start(add=True)
        for d in sc_descs(0):
            d.wait()

    plsc.subcore_barrier()

    pltpu.sync_copy(acc_re.at[pl.ds(r0, STRIPE)], reb)
    pltpu.sync_copy(acc_im.at[pl.ds(r0, STRIPE)], imb)

    @pl.when(cid == 0)
    def _():
        pltpu.sync_copy(reb, o00.at[pl.ds(r0, STRIPE)])
        pltpu.sync_copy(imb, o01.at[pl.ds(r0, STRIPE)])

    @pl.when(cid == 1)
    def _():
        pltpu.sync_copy(reb, o10.at[pl.ds(r0, STRIPE)])
        pltpu.sync_copy(imb, o11.at[pl.ds(r0, STRIPE)])


def _final_body(o00_ref, o01_ref, o10_ref, o11_ref, vre_ref, vim_ref,
                tre_ref, tim_ref, m_ref, out_ref):
    ire = o00_ref[...] + o10_ref[...]
    iim = o01_ref[...] + o11_ref[...]
    vre = vre_ref[...]
    vim = vim_ref[...]
    sre = vre * ire + vim * iim
    sim = vim * ire - vre * iim
    rre = sre - tre_ref[...]
    rim = sim - tim_ref[...]
    m = m_ref[...]
    rre = jnp.where(m, rre, 0.0)
    rim = jnp.where(m, rim, 0.0)
    a = jnp.sqrt(rre * rre + rim * rim)
    l0 = jnp.sum(a)
    l1 = jnp.sum(jnp.abs(rre))
    l2 = jnp.sum(jnp.abs(rim))
    lane = lax.broadcasted_iota(jnp.int32, (1, 128), 1)
    row = jnp.where(lane == 0, l0, jnp.where(lane == 1, l1,
                    jnp.where(lane == 2, l2, 0.0)))
    out_ref[...] = row * (1.0 / N)


_sc_call = pl.kernel(
    _sc_body,
    out_type=[jax.ShapeDtypeStruct((NP,), jnp.float32) for _ in range(4)],
    mesh=plsc.VectorSubcoreMesh(core_axis_name="c", subcore_axis_name="s",
                                num_cores=2, num_subcores=16),
    compiler_params=pltpu.CompilerParams(needs_layout_passes=False),
    scratch_types=[
        pltpu.VMEM((NP,), jnp.float32),
        pltpu.VMEM((NP,), jnp.float32),
        pltpu.VMEM((8, 128), jnp.int32),
        pltpu.VMEM((8, 128), jnp.float32),
        pltpu.VMEM((8, 128), jnp.int32),
        pltpu.VMEM((8, 128), jnp.float32),
        pltpu.VMEM((CB, 128), jnp.int32),
        pltpu.VMEM((CB, 128), jnp.int32),
        pltpu.VMEM((CB, 128), jnp.float32),
        pltpu.VMEM((CB, 128), jnp.float32),
        pltpu.VMEM((CB, 128), jnp.float32),
        pltpu.VMEM((CB, 128), jnp.float32),
        pltpu.VMEM((CB, 128), jnp.int32),
        pltpu.VMEM((CB, 128), jnp.int32),
        pltpu.VMEM((CB, 128), jnp.float32),
        pltpu.VMEM((CB, 128), jnp.float32),
        pltpu.VMEM((CB, 128), jnp.float32),
        pltpu.VMEM((CB, 128), jnp.float32),
        pltpu.VMEM((STRIPE,), jnp.float32),
        pltpu.VMEM((STRIPE,), jnp.float32),
        pltpu.VMEM_SHARED((NP,), jnp.float32),
        pltpu.VMEM_SHARED((NP,), jnp.float32),
        pltpu.SemaphoreType.DMA,
        pltpu.SemaphoreType.DMA,
        pltpu.SemaphoreType.DMA,
        pltpu.SemaphoreType.DMA,
        pltpu.SemaphoreType.DMA,
    ],
)


def kernel(pred, target, edge_index, edge_attr, mask):
    pad = (0, NP - N)
    vm = jnp.pad(pred[:, 0], pad).reshape(1, NP)
    va = jnp.pad(pred[:, 1], pad).reshape(1, NP)
    tre = jnp.pad(target[:, 0], pad).reshape(1, NP)
    tim = jnp.pad(target[:, 1], pad).reshape(1, NP)
    mp = jnp.pad(mask, pad).reshape(1, NP)
    ei2 = (edge_index.reshape(2, E // 128, 128)
           .transpose(1, 0, 2).reshape(E // 64, 128))
    ea2 = (edge_attr.reshape(E // 128, 128, 2)
           .transpose(0, 2, 1).reshape(E // 64, 128))
    zsm = jnp.zeros((STRIPE,), jnp.float32)

    vre_h, vim_h = pl.pallas_call(
        _prep_body,
        out_shape=[jax.ShapeDtypeStruct((1, NP), jnp.float32)] * 2,
    )(vm, va)

    o00, o01, o10, o11 = _sc_call(ei2, ea2, vre_h, vim_h, zsm)

    out = pl.pallas_call(
        _final_body,
        out_shape=jax.ShapeDtypeStruct((1, 128), jnp.float32),
    )(o00.reshape(1, NP), o01.reshape(1, NP), o10.reshape(1, NP),
      o11.reshape(1, NP), vre_h, vim_h, tre, tim, mp)
    return out[0, :3]


# speedup vs baseline: 2720.5256x; 1.0049x over previous
import jax
import jax.numpy as jnp
from jax import lax
from jax.experimental import pallas as pl
from jax.experimental.pallas import tpu as pltpu
from jax.experimental.pallas import tpu_sc as plsc

N = 50000
E = 1600000
NP = 50176
STRIPE = NP // 16
CB = 4
C = CB * 128
W = 32
NCHUNK = E // C
CHUNK_BASE = NCHUNK // W
CHUNK_REM = NCHUNK % W


def _prep_body(vm_ref, va_ref, vre_ref, vim_ref):
    vm = vm_ref[...]
    va = va_ref[...]
    vre_ref[...] = vm * jnp.cos(va)
    vim_ref[...] = vm * jnp.sin(va)


def _sc_body(ei2_h, ea2_h, vre_h, vim_h, zsm,
             o00, o01, o10, o11,
             vre, vim,
             eib_0, attrb_0,
             eib_1, attrb_1,
             srcb2_0, dstb2_0, csr_0, csi_0, cdr_0, cdi_0,
             srcb2_1, dstb2_1, csr_1, csi_1, cdr_1, cdi_1,
             reb, imb, acc_re, acc_im,
             vsem, isem0, isem1, ssem0, ssem1):
    cid = lax.axis_index("c")
    sid = lax.axis_index("s")
    w = cid * 16 + sid
    r0 = sid * STRIPE
    iota = lax.iota(jnp.int32, 16)
    chunk0 = w * CHUNK_BASE

    INB = [(eib_0, attrb_0), (eib_1, attrb_1)]
    STG = [(srcb2_0, dstb2_0, csr_0, csi_0, cdr_0, cdi_0),
           (srcb2_1, dstb2_1, csr_1, csi_1, cdr_1, cdi_1)]
    ISEM = [isem0, isem1]
    SSEM = [ssem0, ssem1]

    def in_descs(g, p):
        eb, ab = INB[p]
        return [
            pltpu.make_async_copy(ei2_h.at[pl.ds(g * 8, 8)], eb, ISEM[p]),
            pltpu.make_async_copy(ea2_h.at[pl.ds(g * 8, 8)], ab, ISEM[p]),
        ]

    def sc_descs(p):
        s2, d2, cr, ci, dr_, di_ = STG[p]
        return [
            pltpu.make_async_copy(cr, acc_re.at[s2], SSEM[p]),
            pltpu.make_async_copy(ci, acc_im.at[s2], SSEM[p]),
            pltpu.make_async_copy(dr_, acc_re.at[d2], SSEM[p]),
            pltpu.make_async_copy(di_, acc_im.at[d2], SSEM[p]),
        ]

    def compute(p):
        eb, ab = INB[p]
        s2, d2, cr, ci, dr_, di_ = STG[p]

        @plsc.parallel_loop(0, C // 16)
        def j_body(t):
            blk = 2 * (t // 8)
            loff = (t % 8) * 16
            s = eb[blk, pl.ds(loff, 16)]
            d = eb[blk + 1, pl.ds(loff, 16)]
            yre = ab[blk, pl.ds(loff, 16)]
            yim = ab[blk + 1, pl.ds(loff, 16)]
            vsr = plsc.load_gather(vre, [s])
            vsi = plsc.load_gather(vim, [s])
            vdr = plsc.load_gather(vre, [d])
            vdi = plsc.load_gather(vim, [d])
            dre = vsr - vdr
            dim = vsi - vdi
            cre = yre * dre - yim * dim
            cim = yre * dim + yim * dre
            off = t * 16
            s2[pl.ds(off, 16)] = s
            d2[pl.ds(off, 16)] = d
            cr[pl.ds(off, 16)] = cre
            ci[pl.ds(off, 16)] = cim
            dr_[pl.ds(off, 16)] = -cre
            di_[pl.ds(off, 16)] = -cim

    vdesc = [pltpu.make_async_copy(vre_h.at[0], vre, vsem),
             pltpu.make_async_copy(vim_h.at[0], vim, vsem)]
    for d in vdesc:
        d.start()
    for d in in_descs(chunk0, 0):
        d.start()
    for d in in_descs(chunk0 + 1, 1):
        d.start()
    pltpu.sync_copy(zsm, reb)
    pltpu.sync_copy(reb, acc_re.at[pl.ds(r0, STRIPE)])
    pltpu.sync_copy(reb, acc_im.at[pl.ds(r0, STRIPE)])
    for d in vdesc:
        d.wait()
    plsc.subcore_barrier()

    def phase(L, p):
        for d in in_descs(chunk0 + L, p):
            d.wait()

        @pl.when(L >= 2)
        def _():
            for d in sc_descs(p):
                d.wait()

        compute(p)
        for d in sc_descs(p):
            d.start(add=True)

        @pl.when(L + 2 < CHUNK_BASE)
        def _():
            for d in in_descs(chunk0 + L + 2, p):
                d.start()

    def body2(k2, carry):
        phase(2 * k2, 0)
        phase(2 * k2 + 1, 1)
        return carry

    lax.fori_loop(0, CHUNK_BASE // 2, body2, 0)
    if CHUNK_BASE % 2:
        phase(jnp.int32(CHUNK_BASE - 1), 0)
    for d in sc_descs(0):
        d.wait()
    for d in sc_descs(1):
        d.wait()

    @pl.when(w < CHUNK_REM)
    def _():
        g = W * CHUNK_BASE + w
        for d in in_descs(g, 0):
            d.start()
        for d in in_descs(g, 0):
            d.wait()
        compute(0)
        for d in sc_descs(0):
            d.start(add=True)
        for d in sc_descs(0):
            d.wait()

    plsc.subcore_barrier()

    pltpu.sync_copy(acc_re.at[pl.ds(r0, STRIPE)], reb)
    pltpu.sync_copy(acc_im.at[pl.ds(r0, STRIPE)], imb)

    @pl.when(cid == 0)
    def _():
        pltpu.sync_copy(reb, o00.at[pl.ds(r0, STRIPE)])
        pltpu.sync_copy(imb, o01.at[pl.ds(r0, STRIPE)])

    @pl.when(cid == 1)
    def _():
        pltpu.sync_copy(reb, o10.at[pl.ds(r0, STRIPE)])
        pltpu.sync_copy(imb, o11.at[pl.ds(r0, STRIPE)])


def _final_body(o00_ref, o01_ref, o10_ref, o11_ref, vre_ref, vim_ref,
                tre_ref, tim_ref, m_ref, out_ref):
    ire = o00_ref[...] + o10_ref[...]
    iim = o01_ref[...] + o11_ref[...]
    vre = vre_ref[...]
    vim = vim_ref[...]
    sre = vre * ire + vim * iim
    sim = vim * ire - vre * iim
    rre = sre - tre_ref[...]
    rim = sim - tim_ref[...]
    m = m_ref[...]
    rre = jnp.where(m, rre, 0.0)
    rim = jnp.where(m, rim, 0.0)
    a = jnp.sqrt(rre * rre + rim * rim)
    l0 = jnp.sum(a)
    l1 = jnp.sum(jnp.abs(rre))
    l2 = jnp.sum(jnp.abs(rim))
    lane = lax.broadcasted_iota(jnp.int32, (1, 128), 1)
    row = jnp.where(lane == 0, l0, jnp.where(lane == 1, l1,
                    jnp.where(lane == 2, l2, 0.0)))
    out_ref[...] = row * (1.0 / N)


_sc_call = pl.kernel(
    _sc_body,
    out_type=[jax.ShapeDtypeStruct((NP,), jnp.float32) for _ in range(4)],
    mesh=plsc.VectorSubcoreMesh(core_axis_name="c", subcore_axis_name="s",
                                num_cores=2, num_subcores=16),
    compiler_params=pltpu.CompilerParams(needs_layout_passes=False),
    scratch_types=[
        pltpu.VMEM((NP,), jnp.float32),
        pltpu.VMEM((NP,), jnp.float32),
        pltpu.VMEM((8, 128), jnp.int32),
        pltpu.VMEM((8, 128), jnp.float32),
        pltpu.VMEM((8, 128), jnp.int32),
        pltpu.VMEM((8, 128), jnp.float32),
        pltpu.VMEM((C,), jnp.int32),
        pltpu.VMEM((C,), jnp.int32),
        pltpu.VMEM((C,), jnp.float32),
        pltpu.VMEM((C,), jnp.float32),
        pltpu.VMEM((C,), jnp.float32),
        pltpu.VMEM((C,), jnp.float32),
        pltpu.VMEM((C,), jnp.int32),
        pltpu.VMEM((C,), jnp.int32),
        pltpu.VMEM((C,), jnp.float32),
        pltpu.VMEM((C,), jnp.float32),
        pltpu.VMEM((C,), jnp.float32),
        pltpu.VMEM((C,), jnp.float32),
        pltpu.VMEM((STRIPE,), jnp.float32),
        pltpu.VMEM((STRIPE,), jnp.float32),
        pltpu.VMEM_SHARED((NP,), jnp.float32),
        pltpu.VMEM_SHARED((NP,), jnp.float32),
        pltpu.SemaphoreType.DMA,
        pltpu.SemaphoreType.DMA,
        pltpu.SemaphoreType.DMA,
        pltpu.SemaphoreType.DMA,
        pltpu.SemaphoreType.DMA,
    ],
)


def kernel(pred, target, edge_index, edge_attr, mask):
    pad = (0, NP - N)
    vm = jnp.pad(pred[:, 0], pad).reshape(1, NP)
    va = jnp.pad(pred[:, 1], pad).reshape(1, NP)
    tre = jnp.pad(target[:, 0], pad).reshape(1, NP)
    tim = jnp.pad(target[:, 1], pad).reshape(1, NP)
    mp = jnp.pad(mask, pad).reshape(1, NP)
    ei2 = (edge_index.reshape(2, E // 128, 128)
           .transpose(1, 0, 2).reshape(E // 64, 128))
    ea2 = (edge_attr.reshape(E // 128, 128, 2)
           .transpose(0, 2, 1).reshape(E // 64, 128))
    zsm = jnp.zeros((STRIPE,), jnp.float32)

    vre_h, vim_h = pl.pallas_call(
        _prep_body,
        out_shape=[jax.ShapeDtypeStruct((1, NP), jnp.float32)] * 2,
    )(vm, va)

    o00, o01, o10, o11 = _sc_call(ei2, ea2, vre_h, vim_h, zsm)

    out = pl.pallas_call(
        _final_body,
        out_shape=jax.ShapeDtypeStruct((1, 128), jnp.float32),
    )(o00.reshape(1, NP), o01.reshape(1, NP), o10.reshape(1, NP),
      o11.reshape(1, NP), vre_h, vim_h, tre, tim, mp)
    return out[0, :3]
